# baseline (speedup 1.0000x reference)
import jax
import jax.numpy as jnp
from jax import lax
from jax.experimental import pallas as pl
from jax.experimental.pallas import tpu as pltpu
from jax.experimental.pallas import tpu_sc as plsc

NW_SPIXELS = 32
NH_SPIXELS = 32
K = NW_SPIXELS * NH_SPIXELS
LANES = 16
NUM_CORES = 2
NUM_SUBCORES = 16
NUM_WORKERS = NUM_CORES * NUM_SUBCORES

CHUNK = 1024
PAD = LANES


NBUF = 4


def _sc_calc_assoc(pf_hbm, sf_hbm, im_hbm, out_hbm, ptab_v, idx_v,
                   pix_v, out_v, isem0, isem1, isem2, isem3,
                   psem0, psem1, psem2, psem3, osem0, osem1):
  B, C, HW = pf_hbm.shape
  workers_per_batch = NUM_WORKERS // B
  per_worker = HW // workers_per_batch
  n_chunks = per_worker // CHUNK

  isems = (isem0, isem1, isem2, isem3)
  psems = (psem0, psem1, psem2, psem3)
  osems = (osem0, osem1)

  wid = lax.axis_index("s") * NUM_CORES + lax.axis_index("c")
  b = wid // workers_per_batch
  base = (wid % workers_per_batch) * per_worker

  pltpu.sync_copy(sf_hbm.at[b], pix_v.at[0, :, pl.ds(0, K)])

  def pack_body(i, carry):
    s = pl.multiple_of(i * LANES, LANES)
    for cp in range(C // 2):
      a = pix_v[0, 2 * cp, pl.ds(s, LANES)]
      bb = pix_v[0, 2 * cp + 1, pl.ds(s, LANES)]
      packed = plsc.pack(a, bb, format=plsc.PackFormat.INTERLEAVED)
      ptab_v[cp, pl.ds(s, LANES)] = plsc.bitcast(packed, jnp.int32)
    return carry

  lax.fori_loop(0, K // LANES, pack_body, 0)

  def in_copies(ci, k):
    off = base + ci * CHUNK
    return (
        pltpu.make_async_copy(im_hbm.at[b, pl.ds(off, CHUNK)],
                              idx_v.at[k, pl.ds(0, CHUNK)], isems[k]),
        pltpu.make_async_copy(pf_hbm.at[b, :, pl.ds(off, CHUNK)],
                              pix_v.at[k, :, pl.ds(0, CHUNK)], psems[k]),
    )

  def out_copy(ci, k):
    off = base + ci * CHUNK
    return pltpu.make_async_copy(out_v.at[k],
                                 out_hbm.at[b, :, pl.ds(off, CHUNK)], osems[k])

  def start_in(ci, k):
    for cp in in_copies(ci, k):
      cp.start()

  for _ci in range(NBUF - 1):
    start_in(_ci, _ci)

  def compute(ci, k, ko):
    idx_b, pix_b, out_b = idx_v.at[k], pix_v.at[k], out_v.at[ko]

    def prefetch(s):
      idx = idx_b[pl.ds(s, LANES)]
      sx = jnp.bitwise_and(idx, NW_SPIXELS - 1)
      sy = jnp.right_shift(idx, 5)
      nys = (jnp.maximum(sy - 1, 0), sy, jnp.minimum(sy + 1, NH_SPIXELS - 1))
      nxs = (jnp.maximum(sx - 1, 0), sx, jnp.minimum(sx + 1, NW_SPIXELS - 1))
      rows = [jnp.left_shift(ny, 5) for ny in nys]
      nidx = [row + nx for row in rows for nx in nxs]
      pp = []
      for cp in range(C // 2):
        a = pix_b[2 * cp, pl.ds(s, LANES)]
        bb = pix_b[2 * cp + 1, pl.ds(s, LANES)]
        pp.append(plsc.pack(a, bb, format=plsc.PackFormat.INTERLEAVED))
      return tuple(nidx), tuple(pp)

    def vec_body(i, carry2):
      s = pl.multiple_of(i * LANES, LANES)
      nidx, pp = carry2
      nxt = prefetch(s + LANES)
      acc = [None] * 9
      for cp in range(C // 2):
        row_ref = ptab_v.at[cp]
        for n in range(9):
          g = plsc.bitcast(plsc.load_gather(row_ref, [nidx[n]]), jnp.bfloat16)
          t = pp[cp] - g
          t = t * t
          acc[n] = t if acc[n] is None else acc[n] + t
      for n in range(9):
        ua, ub = plsc.unpack(acc[n], format=plsc.PackFormat.INTERLEAVED)
        out_b[n, pl.ds(s, LANES)] = ua + ub
      return nxt

    lax.fori_loop(0, CHUNK // LANES, vec_body, prefetch(0))

  def outer(j, carry):
    ci0 = NBUF * j
    for k in range(NBUF):
      ci = ci0 + k
      ko = k % 2

      @pl.when(ci + NBUF - 1 < n_chunks)
      def _():
        start_in(ci + NBUF - 1, (k + NBUF - 1) % NBUF)

      for cp in in_copies(ci, k):
        cp.wait()

      @pl.when(ci >= 2)
      def _():
        out_copy(ci - 2, ko).wait()

      compute(ci, k, ko)
      out_copy(ci, ko).start()
    return carry

  lax.fori_loop(0, n_chunks // NBUF, outer, 0)
  out_copy(n_chunks - 2, 0).wait()
  out_copy(n_chunks - 1, 1).wait()


def kernel(pixel_feats, spixel_feats, index_map):
  B, C, H, W = pixel_feats.shape
  HW = H * W
  pf = pixel_feats.reshape(B, C, HW)
  im = index_map.reshape(B, HW)

  mesh = plsc.VectorSubcoreMesh(
      core_axis_name="c", subcore_axis_name="s",
      num_cores=NUM_CORES, num_subcores=NUM_SUBCORES)
  run = pl.kernel(
      _sc_calc_assoc,
      out_type=jax.ShapeDtypeStruct((B, 9, HW), jnp.float32),
      mesh=mesh,
      compiler_params=pltpu.CompilerParams(use_tc_tiling_on_sc=False,
                                           needs_layout_passes=False),
      scratch_types=[
          pltpu.VMEM((C // 2, K), jnp.int32),
          pltpu.VMEM((4, CHUNK + PAD), jnp.int32),
          pltpu.VMEM((4, C, CHUNK + PAD), jnp.float32),
          pltpu.VMEM((2, 9, CHUNK), jnp.float32),
      ] + [pltpu.SemaphoreType.DMA] * 10,
  )
  out = run(pf, spixel_feats, im)
  return out.reshape(B, 9, H, W)


# speedup vs baseline: 1.0109x; 1.0109x over previous
import jax
import jax.numpy as jnp
from jax import lax
from jax.experimental import pallas as pl
from jax.experimental.pallas import tpu as pltpu
from jax.experimental.pallas import tpu_sc as plsc

NW_SPIXELS = 32
NH_SPIXELS = 32
K = NW_SPIXELS * NH_SPIXELS
LANES = 16
NUM_CORES = 2
NUM_SUBCORES = 16
NUM_WORKERS = NUM_CORES * NUM_SUBCORES

TILE_R = 8
TILE_C = 128
TPX = TILE_R * TILE_C


def _sc_calc_assoc(pf_hbm, ptab_hbm, im_hbm, out_hbm, ptab_v, idx_v, pix_v,
                   out_v, isem0, isem1, psem0, psem1, osem0, osem1):
  B, C, NT = pf_hbm.shape[0], pf_hbm.shape[1], pf_hbm.shape[2]
  workers_per_batch = NUM_WORKERS // B
  n_chunks = NT // workers_per_batch

  isems = (isem0, isem1)
  psems = (psem0, psem1)
  osems = (osem0, osem1)

  wid = lax.axis_index("s") * NUM_CORES + lax.axis_index("c")
  b = wid // workers_per_batch
  base_t = (wid % workers_per_batch) * n_chunks

  pltpu.sync_copy(ptab_hbm.at[b], ptab_v)

  def in_copies(ci, k):
    t = base_t + ci
    return (
        pltpu.make_async_copy(im_hbm.at[b, t], idx_v.at[k], isems[k]),
        pltpu.make_async_copy(pf_hbm.at[b, :, t], pix_v.at[k], psems[k]),
    )

  def out_copy(ci, k):
    t = base_t + ci
    return pltpu.make_async_copy(out_v.at[k], out_hbm.at[b, :, t], osems[k])

  def start_in(ci, k):
    for cp in in_copies(ci, k):
      cp.start()

  start_in(0, 0)

  def compute(ci, k):
    idx_b, pix_b, out_b = idx_v.at[k], pix_v.at[k], out_v.at[k]
    n_vecs = TPX // LANES

    def addr(i):
      r = jnp.right_shift(i, 3)
      col = pl.multiple_of(
          jnp.left_shift(jnp.bitwise_and(i, 7), 4), LANES)
      return r, col

    def prefetch(i):
      r, col = addr(i)
      idx = idx_b[r, pl.ds(col, LANES)]
      sx = jnp.bitwise_and(idx, NW_SPIXELS - 1)
      sy = jnp.right_shift(idx, 5)
      nys = (jnp.maximum(sy - 1, 0), sy, jnp.minimum(sy + 1, NH_SPIXELS - 1))
      nxs = (jnp.maximum(sx - 1, 0), sx, jnp.minimum(sx + 1, NW_SPIXELS - 1))
      rows = [jnp.left_shift(ny, 5) for ny in nys]
      nidx = [row + nx for row in rows for nx in nxs]
      pp = []
      for cp in range(C // 2):
        a = pix_b[2 * cp, r, pl.ds(col, LANES)]
        bb = pix_b[2 * cp + 1, r, pl.ds(col, LANES)]
        pp.append(plsc.pack(a, bb, format=plsc.PackFormat.INTERLEAVED))
      return tuple(nidx), tuple(pp)

    def vec_body(i, carry2):
      r, col = addr(i)
      nidx, pp = carry2
      nxt = prefetch(jnp.minimum(i + 1, n_vecs - 1))
      acc = [None] * 9
      for cp in range(C // 2):
        row_ref = ptab_v.at[cp]
        for n in range(9):
          g = plsc.bitcast(plsc.load_gather(row_ref, [nidx[n]]), jnp.bfloat16)
          t = pp[cp] - g
          t = t * t
          acc[n] = t if acc[n] is None else acc[n] + t
      for n in range(9):
        ua, ub = plsc.unpack(acc[n], format=plsc.PackFormat.INTERLEAVED)
        out_b[n, r, pl.ds(col, LANES)] = ua + ub
      return nxt

    lax.fori_loop(0, n_vecs, vec_body, prefetch(0))

  def outer(j, carry):
    ci0 = 2 * j
    for k in (0, 1):
      ci = ci0 + k

      @pl.when(ci + 1 < n_chunks)
      def _():
        start_in(ci + 1, 1 - k)

      for cp in in_copies(ci, k):
        cp.wait()

      @pl.when(ci >= 2)
      def _():
        out_copy(ci - 2, k).wait()

      compute(ci, k)
      out_copy(ci, k).start()
    return carry

  lax.fori_loop(0, n_chunks // 2, outer, 0)
  out_copy(n_chunks - 2, 0).wait()
  out_copy(n_chunks - 1, 1).wait()


def kernel(pixel_feats, spixel_feats, index_map):
  B, C, H, W = pixel_feats.shape
  NT = (H // TILE_R) * (W // TILE_C)

  pf = pixel_feats.reshape(B, C, NT, TILE_R, TILE_C)
  im = index_map.reshape(B, NT, TILE_R, TILE_C)

  sb = spixel_feats.astype(jnp.bfloat16)
  lo = lax.bitcast_convert_type(sb[:, 0::2, :], jnp.uint16).astype(jnp.uint32)
  hi = lax.bitcast_convert_type(sb[:, 1::2, :], jnp.uint16).astype(jnp.uint32)
  ptab = lax.bitcast_convert_type(lo | (hi << 16), jnp.int32)

  mesh = plsc.VectorSubcoreMesh(
      core_axis_name="c", subcore_axis_name="s",
      num_cores=NUM_CORES, num_subcores=NUM_SUBCORES)
  run = pl.kernel(
      _sc_calc_assoc,
      out_type=jax.ShapeDtypeStruct((B, 9, NT, TILE_R, TILE_C), jnp.float32),
      mesh=mesh,
      compiler_params=pltpu.CompilerParams(use_tc_tiling_on_sc=False,
                                           needs_layout_passes=False),
      scratch_types=[
          pltpu.VMEM((C // 2, K), jnp.int32),
          pltpu.VMEM((2, TILE_R, TILE_C), jnp.int32),
          pltpu.VMEM((2, C, TILE_R, TILE_C), jnp.float32),
          pltpu.VMEM((2, 9, TILE_R, TILE_C), jnp.float32),
      ] + [pltpu.SemaphoreType.DMA] * 6,
  )
  out = run(pf, ptab, im)
  return out.reshape(B, 9, H, W)


# speedup vs baseline: 1.4425x; 1.4269x over previous
import jax
import jax.numpy as jnp
from jax import lax
from jax.experimental import pallas as pl
from jax.experimental.pallas import tpu as pltpu
from jax.experimental.pallas import tpu_sc as plsc

NW_SPIXELS = 32
NH_SPIXELS = 32
K = NW_SPIXELS * NH_SPIXELS
LANES = 16
NUM_CORES = 2
NUM_SUBCORES = 16
NUM_WORKERS = NUM_CORES * NUM_SUBCORES

TILE_R = 8
TILE_C = 128
TPX = TILE_R * TILE_C


def _sc_calc_assoc(pf_hbm, ptab_hbm, im_hbm, out_hbm, ptab_v, idx_v, pix_v,
                   out_v, isem0, isem1, psem0, psem1, osem0, osem1):
  B, C, NT = pf_hbm.shape[0], pf_hbm.shape[1], pf_hbm.shape[2]
  workers_per_batch = NUM_WORKERS // B
  n_chunks = NT // workers_per_batch

  isems = (isem0, isem1)
  psems = (psem0, psem1)
  osems = (osem0, osem1)

  wid = lax.axis_index("s") * NUM_CORES + lax.axis_index("c")
  b = wid // workers_per_batch
  base_t = (wid % workers_per_batch) * n_chunks

  pltpu.sync_copy(ptab_hbm.at[b], ptab_v)

  def in_copies(ci, k):
    t = base_t + ci
    return (
        pltpu.make_async_copy(im_hbm.at[b, t], idx_v.at[k], isems[k]),
        pltpu.make_async_copy(pf_hbm.at[b, :, t], pix_v.at[k], psems[k]),
    )

  def out_copy(ci, k):
    t = base_t + ci
    return pltpu.make_async_copy(out_v.at[k], out_hbm.at[b, :, t], osems[k])

  def start_in(ci, k):
    for cp in in_copies(ci, k):
      cp.start()

  start_in(0, 0)

  def compute(ci, k):
    idx_b, pix_b, out_b = idx_v.at[k], pix_v.at[k], out_v.at[k]
    n_vecs = TPX // LANES

    def addr(i):
      r = jnp.right_shift(i, 3)
      col = pl.multiple_of(
          jnp.left_shift(jnp.bitwise_and(i, 7), 4), LANES)
      return r, col

    def prefetch(i):
      r, col = addr(i)
      idx = idx_b[r, pl.ds(col, LANES)]
      sx = jnp.bitwise_and(idx, NW_SPIXELS - 1)
      sy = jnp.right_shift(idx, 5)
      nys = (jnp.maximum(sy - 1, 0), sy, jnp.minimum(sy + 1, NH_SPIXELS - 1))
      nxs = (jnp.maximum(sx - 1, 0), sx, jnp.minimum(sx + 1, NW_SPIXELS - 1))
      rows = [jnp.left_shift(ny, 5) for ny in nys]
      nidx = [row + nx for row in rows for nx in nxs]
      pp = []
      for cp in range(C // 2):
        a = pix_b[2 * cp, r, pl.ds(col, LANES)]
        bb = pix_b[2 * cp + 1, r, pl.ds(col, LANES)]
        pp.append(plsc.pack(a, bb, format=plsc.PackFormat.INTERLEAVED))
      return tuple(nidx), tuple(pp)

    def vec_body(i, carry2):
      r, col = addr(i)
      nidx, pp = carry2
      nxt = prefetch(jnp.minimum(i + 1, n_vecs - 1))
      acc = [None] * 9
      for cp in range(C // 2):
        row_ref = ptab_v.at[cp]
        for n in range(9):
          g = plsc.bitcast(plsc.load_gather(row_ref, [nidx[n]]), jnp.bfloat16)
          t = pp[cp] - g
          t = t * t
          acc[n] = t if acc[n] is None else acc[n] + t
      for n in range(9):
        ua, ub = plsc.unpack(acc[n], format=plsc.PackFormat.INTERLEAVED)
        out_b[n, r, pl.ds(col, LANES)] = ua + ub
      return nxt

    lax.fori_loop(0, n_vecs, vec_body, prefetch(0))

  def outer(j, carry):
    ci0 = 2 * j
    for k in (0, 1):
      ci = ci0 + k

      @pl.when(ci + 1 < n_chunks)
      def _():
        start_in(ci + 1, 1 - k)

      for cp in in_copies(ci, k):
        cp.wait()

      @pl.when(ci >= 2)
      def _():
        out_copy(ci - 2, k).wait()

      compute(ci, k)
      out_copy(ci, k).start()
    return carry

  lax.fori_loop(0, n_chunks // 2, outer, 0)
  out_copy(n_chunks - 2, 0).wait()
  out_copy(n_chunks - 1, 1).wait()


def kernel(pixel_feats, spixel_feats, index_map):
  B, C, H, W = pixel_feats.shape
  NT = (H // TILE_R) * (W // TILE_C)

  HB, WT = H // TILE_R, W // TILE_C
  pf = (pixel_feats.reshape(B, C, HB, TILE_R, WT, TILE_C)
        .transpose(0, 1, 2, 4, 3, 5).reshape(B, C, NT, TILE_R, TILE_C))
  im = (index_map.reshape(B, HB, TILE_R, WT, TILE_C)
        .transpose(0, 1, 3, 2, 4).reshape(B, NT, TILE_R, TILE_C))

  sb = spixel_feats.astype(jnp.bfloat16)
  lo = lax.bitcast_convert_type(sb[:, 0::2, :], jnp.uint16).astype(jnp.uint32)
  hi = lax.bitcast_convert_type(sb[:, 1::2, :], jnp.uint16).astype(jnp.uint32)
  ptab = lax.bitcast_convert_type(lo | (hi << 16), jnp.int32)

  mesh = plsc.VectorSubcoreMesh(
      core_axis_name="c", subcore_axis_name="s",
      num_cores=NUM_CORES, num_subcores=NUM_SUBCORES)
  run = pl.kernel(
      _sc_calc_assoc,
      out_type=jax.ShapeDtypeStruct((B, 9, NT, TILE_R, TILE_C), jnp.float32),
      mesh=mesh,
      compiler_params=pltpu.CompilerParams(use_tc_tiling_on_sc=False,
                                           needs_layout_passes=False),
      scratch_types=[
          pltpu.VMEM((C // 2, K), jnp.int32),
          pltpu.VMEM((2, TILE_R, TILE_C), jnp.int32),
          pltpu.VMEM((2, C, TILE_R, TILE_C), jnp.float32),
          pltpu.VMEM((2, 9, TILE_R, TILE_C), jnp.float32),
      ] + [pltpu.SemaphoreType.DMA] * 6,
  )
  out = run(pf, ptab, im)
  out = (out.reshape(B, 9, HB, WT, TILE_R, TILE_C)
         .transpose(0, 1, 2, 4, 3, 5).reshape(B, 9, H, W))
  return out
